# Initial kernel scaffold; baseline (speedup 1.0000x reference)
import jax
import jax.numpy as jnp
from jax.experimental import pallas as pl


def kernel(x, y, Wc, bc, Wih, Whh, bih, bhh, Wfc, bfc):
    raise NotImplementedError("write your pallas kernel here")


# speedup vs baseline: 64.6350x; 64.6350x over previous
import functools

import jax
import jax.numpy as jnp
from jax.experimental import pallas as pl
from jax.experimental.pallas import tpu as pltpu

WS = 7
NN = 10
B, C, T, F = 16, 3, 448, 56
H = T // WS
TS = B * C * H
NSEQ = B * H
GDIM = 3 * F
HID = 56
NORM = 1.0 / (NN * F * H * C)


def _cmpx(a, b):
    return jnp.minimum(a, b), jnp.maximum(a, b)


def _median7(v):
    v = list(v)
    for r in range(7):
        pairs = ((0, 1), (2, 3), (4, 5)) if r % 2 == 0 else ((1, 2), (3, 4), (5, 6))
        for i, j in pairs:
            v[i], v[j] = _cmpx(v[i], v[j])
    return v[3]


def _stats_kernel(x_ref, o_ref):
    w = [x_ref[k] for k in range(WS)]
    amax = w[0]
    amin = w[0]
    s = w[0]
    for k in range(1, WS):
        amax = jnp.maximum(amax, w[k])
        amin = jnp.minimum(amin, w[k])
        s = s + w[k]
    mu = s * (1.0 / WS)
    ssd = (w[0] - mu) ** 2
    for k in range(1, WS):
        ssd = ssd + (w[k] - mu) ** 2
    astd = jnp.sqrt(ssd * (1.0 / (WS - 1)))
    sd0 = jnp.sqrt(ssd * (1.0 / WS))
    inv_sd = 1.0 / sd0
    z = [(w[k] - mu) * inv_sd for k in range(WS)]
    zm = z[0]
    for k in range(1, WS):
        zm = zm + z[k]
    zm = zm * (1.0 / WS)
    d = [z[k] - zm for k in range(WS)]
    m2 = d[0] * d[0]
    m3 = d[0] * d[0] * d[0]
    m4 = (d[0] * d[0]) * (d[0] * d[0])
    for k in range(1, WS):
        dk2 = d[k] * d[k]
        m2 = m2 + dk2
        m3 = m3 + dk2 * d[k]
        m4 = m4 + dk2 * dk2
    m2 = m2 * (1.0 / WS)
    m3 = m3 * (1.0 / WS)
    m4 = m4 * (1.0 / WS)
    skew = m3 / jnp.power(m2, 1.5)
    kurt = m4 / (m2 * m2) - 3.0
    med = _median7(w)
    mad = _median7([jnp.abs(w[k] - med) for k in range(WS)])
    o_ref[0] = amax
    o_ref[1] = amin
    o_ref[2] = mu
    o_ref[3] = astd
    o_ref[4] = skew
    o_ref[5] = kurt
    o_ref[6] = mad


def _relieff_kernel(xr_ref, xc_ref, o_ref):
    tb = xr_ref.shape[0]
    i0 = pl.program_id(0)

    dist = jnp.zeros((tb, F, F), dtype=jnp.float32)
    for d in range(WS):
        diff = xr_ref[:, :, d:d + 1] - xc_ref[:, d:d + 1, :]
        dist = dist + diff * diff

    big = jnp.float32(3.4e38)
    work = dist
    v10 = None
    v20 = None
    for r in range(2 * NN):
        m = jnp.min(work, axis=-1, keepdims=True)
        if r == NN - 1:
            v10 = m
        if r == 2 * NN - 1:
            v20 = m
        if r != 2 * NN - 1:
            work = jnp.where(work <= m, big, work)

    mmask = (dist <= v20).astype(jnp.float32) - 2.0 * (dist <= v10).astype(jnp.float32)

    rows = jax.lax.broadcasted_iota(jnp.int32, (8, 128), 0)
    contrib = jnp.zeros((8, 128), dtype=jnp.float32)
    for d in range(WS):
        diff = xr_ref[:, :, d:d + 1] - xc_ref[:, d:d + 1, :]
        sd = jnp.sum(mmask * jnp.abs(diff))
        contrib = jnp.where(rows == d, contrib + sd, contrib)

    @pl.when(i0 == 0)
    def _():
        o_ref[...] = jnp.zeros_like(o_ref)

    o_ref[...] += contrib

    @pl.when(i0 == pl.num_programs(0) - 1)
    def _():
        o_ref[...] = o_ref[...] * NORM


def _gru_kernel(x_ref, wr_ref, wz_ref, wn_ref, br_ref, bz_ref, bn_ref,
                whr_ref, whz_ref, whn_ref, bhr_ref, bhz_ref, bhn_ref,
                wfc_ref, bfc_ref, sc_ref, o_ref, gir, giz, gin):
    x = x_ref[...]
    f32 = jnp.float32
    gir[...] = jnp.dot(x, wr_ref[...], preferred_element_type=f32) + br_ref[...]
    giz[...] = jnp.dot(x, wz_ref[...], preferred_element_type=f32) + bz_ref[...]
    gin[...] = jnp.dot(x, wn_ref[...], preferred_element_type=f32) + bn_ref[...]

    weff = sc_ref[0, 0] * wfc_ref[0]
    beff = sc_ref[0, 0] * bfc_ref[0:1, :]
    for d in range(1, WS):
        weff = weff + sc_ref[d, 0] * wfc_ref[d]
        beff = beff + sc_ref[d, 0] * bfc_ref[d:d + 1, :]

    whr = whr_ref[...]
    whz = whz_ref[...]
    whn = whn_ref[...]
    bhr = bhr_ref[...]
    bhz = bhz_ref[...]
    bhn = bhn_ref[...]

    def step(t, h):
        gr = jnp.dot(h, whr, preferred_element_type=f32) + bhr
        gz = jnp.dot(h, whz, preferred_element_type=f32) + bhz
        gn = jnp.dot(h, whn, preferred_element_type=f32) + bhn
        r = jax.nn.sigmoid(gir[pl.ds(t * B, B), :] + gr)
        zg = jax.nn.sigmoid(giz[pl.ds(t * B, B), :] + gz)
        n = jnp.tanh(gin[pl.ds(t * B, B), :] + r * gn)
        h = (1.0 - zg) * n + zg * h
        o_ref[pl.ds(t * B, B), :] = (
            jnp.dot(h, weff, preferred_element_type=f32) + beff)
        return h

    h0 = jnp.zeros((B, HID), dtype=f32)
    jax.lax.fori_loop(0, H, step, h0)


@jax.jit
def kernel(x, y, Wc, bc, Wih, Whh, bih, bhh, Wfc, bfc):
    f32 = jnp.float32

    xw = x.reshape(B, C, H, WS, F).transpose(3, 0, 1, 2, 4).reshape(WS, 1344, 128)
    stats = pl.pallas_call(
        _stats_kernel,
        grid=(4,),
        in_specs=[pl.BlockSpec((WS, 336, 128), lambda i: (0, i, 0))],
        out_specs=pl.BlockSpec((WS, 336, 128), lambda i: (0, i, 0)),
        out_shape=jax.ShapeDtypeStruct((WS, 1344, 128), f32),
    )(xw)

    ext_s = stats.reshape(WS, TS, F)
    xrow = ext_s.transpose(1, 2, 0)
    xcol = ext_s.transpose(1, 0, 2)

    TB = 16
    scores_pad = pl.pallas_call(
        _relieff_kernel,
        grid=(TS // TB,),
        in_specs=[
            pl.BlockSpec((TB, F, WS), lambda i: (i, 0, 0)),
            pl.BlockSpec((TB, WS, F), lambda i: (i, 0, 0)),
        ],
        out_specs=pl.BlockSpec((8, 128), lambda i: (0, 0)),
        out_shape=jax.ShapeDtypeStruct((8, 128), f32),
    )(xrow, xcol)

    Wih4 = Wih.reshape(GDIM, C, F, WS)
    Wih_eff = jnp.einsum('gcwd,ck->gkwd', Wih4, Wc).reshape(GDIM, C * F * WS)
    bih_eff = bih + jnp.einsum('gcwd,c->g', Wih4, bc)
    wr = Wih_eff[:HID].T
    wz = Wih_eff[HID:2 * HID].T
    wn = Wih_eff[2 * HID:].T
    br = bih_eff[:HID].reshape(1, HID)
    bz = bih_eff[HID:2 * HID].reshape(1, HID)
    bn = bih_eff[2 * HID:].reshape(1, HID)
    whr = Whh[:HID].T
    whz = Whh[HID:2 * HID].T
    whn = Whh[2 * HID:].T
    bhr = bhh[:HID].reshape(1, HID)
    bhz = bhh[HID:2 * HID].reshape(1, HID)
    bhn = bhh[2 * HID:].reshape(1, HID)
    wfc_t = Wfc.reshape(GDIM, WS, HID).transpose(1, 2, 0)
    bfc_t = bfc.reshape(GDIM, WS).T

    xflat = xrow.reshape(NSEQ, C * F * WS)
    xtm = xflat.reshape(B, H, C * F * WS).transpose(1, 0, 2).reshape(NSEQ, C * F * WS)

    full = lambda s: pl.BlockSpec(s, lambda: tuple(0 for _ in s))
    out_tm = pl.pallas_call(
        _gru_kernel,
        in_specs=[full((NSEQ, C * F * WS)),
                  full((C * F * WS, HID)), full((C * F * WS, HID)), full((C * F * WS, HID)),
                  full((1, HID)), full((1, HID)), full((1, HID)),
                  full((HID, HID)), full((HID, HID)), full((HID, HID)),
                  full((1, HID)), full((1, HID)), full((1, HID)),
                  full((WS, HID, GDIM)), full((WS, GDIM)), full((8, 128))],
        out_specs=full((NSEQ, GDIM)),
        out_shape=jax.ShapeDtypeStruct((NSEQ, GDIM), f32),
        scratch_shapes=[pltpu.VMEM((NSEQ, HID), f32)] * 3,
    )(xtm, wr, wz, wn, br, bz, bn, whr, whz, whn, bhr, bhz, bhn,
      wfc_t, bfc_t, scores_pad)

    return out_tm.reshape(H, B, C, F).transpose(1, 0, 2, 3)


# speedup vs baseline: 98.4961x; 1.5239x over previous
import functools

import jax
import jax.numpy as jnp
from jax.experimental import pallas as pl
from jax.experimental.pallas import tpu as pltpu

WS = 7
NN = 10
B, C, T, F = 16, 3, 448, 56
H = T // WS
TS = B * C * H
NSEQ = B * H
GDIM = 3 * F
HID = 56
NORM = 1.0 / (NN * F * H * C)


def _cmpx(a, b):
    return jnp.minimum(a, b), jnp.maximum(a, b)


def _median7(v):
    v = list(v)
    for r in range(7):
        pairs = ((0, 1), (2, 3), (4, 5)) if r % 2 == 0 else ((1, 2), (3, 4), (5, 6))
        for i, j in pairs:
            v[i], v[j] = _cmpx(v[i], v[j])
    return v[3]


def _stats_kernel(x_ref, o_ref):
    w = [x_ref[k] for k in range(WS)]
    amax = w[0]
    amin = w[0]
    s = w[0]
    for k in range(1, WS):
        amax = jnp.maximum(amax, w[k])
        amin = jnp.minimum(amin, w[k])
        s = s + w[k]
    mu = s * (1.0 / WS)
    ssd = (w[0] - mu) ** 2
    for k in range(1, WS):
        ssd = ssd + (w[k] - mu) ** 2
    astd = jnp.sqrt(ssd * (1.0 / (WS - 1)))
    sd0 = jnp.sqrt(ssd * (1.0 / WS))
    inv_sd = 1.0 / sd0
    z = [(w[k] - mu) * inv_sd for k in range(WS)]
    zm = z[0]
    for k in range(1, WS):
        zm = zm + z[k]
    zm = zm * (1.0 / WS)
    d = [z[k] - zm for k in range(WS)]
    m2 = d[0] * d[0]
    m3 = d[0] * d[0] * d[0]
    m4 = (d[0] * d[0]) * (d[0] * d[0])
    for k in range(1, WS):
        dk2 = d[k] * d[k]
        m2 = m2 + dk2
        m3 = m3 + dk2 * d[k]
        m4 = m4 + dk2 * dk2
    m2 = m2 * (1.0 / WS)
    m3 = m3 * (1.0 / WS)
    m4 = m4 * (1.0 / WS)
    skew = m3 / jnp.power(m2, 1.5)
    kurt = m4 / (m2 * m2) - 3.0
    med = _median7(w)
    mad = _median7([jnp.abs(w[k] - med) for k in range(WS)])
    o_ref[0] = amax
    o_ref[1] = amin
    o_ref[2] = mu
    o_ref[3] = astd
    o_ref[4] = skew
    o_ref[5] = kurt
    o_ref[6] = mad


def _relieff_kernel(xr_ref, xc_ref, o_ref):
    pb = xr_ref.shape[0]
    i0 = pl.program_id(0)

    absdiff = []
    dist = jnp.zeros((pb, F, 2 * F), dtype=jnp.float32)
    for d in range(WS):
        xa = jnp.broadcast_to(xr_ref[:, :, 2 * d:2 * d + 1], (pb, F, F))
        xb = jnp.broadcast_to(xr_ref[:, :, 2 * d + 1:2 * d + 2], (pb, F, F))
        left = jnp.concatenate([xa, xb], axis=2)
        a = jnp.abs(left - xc_ref[:, d:d + 1, :])
        absdiff.append(a)
        dist = dist + a * a

    big = jnp.float32(3.4e38)
    ii = jax.lax.broadcasted_iota(jnp.int32, (pb, F, 2 * F), 1)
    jj = jax.lax.broadcasted_iota(jnp.int32, (pb, F, 2 * F), 2)
    work = jnp.where(ii == jj % F, big, dist)
    v10 = None
    v20 = None
    for r in range(2 * NN - 1):
        m = jnp.min(work, axis=1, keepdims=True)
        if r == NN - 2:
            v10 = m
        if r == 2 * NN - 2:
            v20 = m
        else:
            work = jnp.where(work <= m, big, work)

    mmask = (dist <= v20).astype(jnp.float32) - 2.0 * (dist <= v10).astype(jnp.float32)

    rows = jax.lax.broadcasted_iota(jnp.int32, (8, 128), 0)
    contrib = jnp.zeros((8, 128), dtype=jnp.float32)
    for d in range(WS):
        sd = jnp.sum(mmask * absdiff[d])
        contrib = jnp.where(rows == d, contrib + sd, contrib)

    @pl.when(i0 == 0)
    def _():
        o_ref[...] = jnp.zeros_like(o_ref)

    o_ref[...] += contrib

    @pl.when(i0 == pl.num_programs(0) - 1)
    def _():
        o_ref[...] = o_ref[...] * NORM


def _gru_kernel(x_ref, wr_ref, wz_ref, wn_ref, br_ref, bz_ref, bn_ref,
                whr_ref, whz_ref, whn_ref, bhr_ref, bhz_ref, bhn_ref,
                wfc_ref, bfc_ref, sc_ref, o_ref, gir, giz, gin):
    x = x_ref[...]
    f32 = jnp.float32
    gir[...] = jnp.dot(x, wr_ref[...], preferred_element_type=f32) + br_ref[...]
    giz[...] = jnp.dot(x, wz_ref[...], preferred_element_type=f32) + bz_ref[...]
    gin[...] = jnp.dot(x, wn_ref[...], preferred_element_type=f32) + bn_ref[...]

    weff = sc_ref[0, 0] * wfc_ref[0]
    beff = sc_ref[0, 0] * bfc_ref[0:1, :]
    for d in range(1, WS):
        weff = weff + sc_ref[d, 0] * wfc_ref[d]
        beff = beff + sc_ref[d, 0] * bfc_ref[d:d + 1, :]

    whr = whr_ref[...]
    whz = whz_ref[...]
    whn = whn_ref[...]
    bhr = bhr_ref[...]
    bhz = bhz_ref[...]
    bhn = bhn_ref[...]

    def step(t, h):
        gr = jnp.dot(h, whr, preferred_element_type=f32) + bhr
        gz = jnp.dot(h, whz, preferred_element_type=f32) + bhz
        gn = jnp.dot(h, whn, preferred_element_type=f32) + bhn
        r = jax.nn.sigmoid(gir[pl.ds(t * B, B), :] + gr)
        zg = jax.nn.sigmoid(giz[pl.ds(t * B, B), :] + gz)
        n = jnp.tanh(gin[pl.ds(t * B, B), :] + r * gn)
        h = (1.0 - zg) * n + zg * h
        o_ref[pl.ds(t * B, B), :] = (
            jnp.dot(h, weff, preferred_element_type=f32) + beff)
        return h

    h0 = jnp.zeros((B, HID), dtype=f32)
    jax.lax.fori_loop(0, H, step, h0)


@jax.jit
def kernel(x, y, Wc, bc, Wih, Whh, bih, bhh, Wfc, bfc):
    f32 = jnp.float32

    xw = x.reshape(B, C, H, WS, F).transpose(3, 0, 1, 2, 4).reshape(WS, 1344, 128)
    stats = pl.pallas_call(
        _stats_kernel,
        grid=(4,),
        in_specs=[pl.BlockSpec((WS, 336, 128), lambda i: (0, i, 0))],
        out_specs=pl.BlockSpec((WS, 336, 128), lambda i: (0, i, 0)),
        out_shape=jax.ShapeDtypeStruct((WS, 1344, 128), f32),
    )(xw)

    ext_s = stats.reshape(WS, TS, F)
    xrow = ext_s.transpose(1, 2, 0)
    xcol = ext_s.transpose(1, 0, 2)

    PB = 32
    xrq = xrow.reshape(TS // 2, 2, F, WS).transpose(0, 2, 3, 1).reshape(TS // 2, F, 2 * WS)
    xcp = xcol.reshape(TS // 2, 2, WS, F).transpose(0, 2, 1, 3).reshape(TS // 2, WS, 2 * F)
    scores_pad = pl.pallas_call(
        _relieff_kernel,
        grid=(TS // 2 // PB,),
        in_specs=[
            pl.BlockSpec((PB, F, 2 * WS), lambda i: (i, 0, 0)),
            pl.BlockSpec((PB, WS, 2 * F), lambda i: (i, 0, 0)),
        ],
        out_specs=pl.BlockSpec((8, 128), lambda i: (0, 0)),
        out_shape=jax.ShapeDtypeStruct((8, 128), f32),
    )(xrq, xcp)

    Wih4 = Wih.reshape(GDIM, C, F, WS)
    Wih_eff = jnp.einsum('gcwd,ck->gkwd', Wih4, Wc).reshape(GDIM, C * F * WS)
    bih_eff = bih + jnp.einsum('gcwd,c->g', Wih4, bc)
    wr = Wih_eff[:HID].T
    wz = Wih_eff[HID:2 * HID].T
    wn = Wih_eff[2 * HID:].T
    br = bih_eff[:HID].reshape(1, HID)
    bz = bih_eff[HID:2 * HID].reshape(1, HID)
    bn = bih_eff[2 * HID:].reshape(1, HID)
    whr = Whh[:HID].T
    whz = Whh[HID:2 * HID].T
    whn = Whh[2 * HID:].T
    bhr = bhh[:HID].reshape(1, HID)
    bhz = bhh[HID:2 * HID].reshape(1, HID)
    bhn = bhh[2 * HID:].reshape(1, HID)
    wfc_t = Wfc.reshape(GDIM, WS, HID).transpose(1, 2, 0)
    bfc_t = bfc.reshape(GDIM, WS).T

    xflat = xrow.reshape(NSEQ, C * F * WS)
    xtm = xflat.reshape(B, H, C * F * WS).transpose(1, 0, 2).reshape(NSEQ, C * F * WS)

    full = lambda s: pl.BlockSpec(s, lambda: tuple(0 for _ in s))
    out_tm = pl.pallas_call(
        _gru_kernel,
        in_specs=[full((NSEQ, C * F * WS)),
                  full((C * F * WS, HID)), full((C * F * WS, HID)), full((C * F * WS, HID)),
                  full((1, HID)), full((1, HID)), full((1, HID)),
                  full((HID, HID)), full((HID, HID)), full((HID, HID)),
                  full((1, HID)), full((1, HID)), full((1, HID)),
                  full((WS, HID, GDIM)), full((WS, GDIM)), full((8, 128))],
        out_specs=full((NSEQ, GDIM)),
        out_shape=jax.ShapeDtypeStruct((NSEQ, GDIM), f32),
        scratch_shapes=[pltpu.VMEM((NSEQ, HID), f32)] * 3,
    )(xtm, wr, wz, wn, br, bz, bn, whr, whz, whn, bhr, bhz, bhn,
      wfc_t, bfc_t, scores_pad)

    return out_tm.reshape(H, B, C, F).transpose(1, 0, 2, 3)


# speedup vs baseline: 120.7191x; 1.2256x over previous
import functools

import jax
import jax.numpy as jnp
from jax.experimental import pallas as pl
from jax.experimental.pallas import tpu as pltpu

WS = 7
NN = 10
B, C, T, F = 16, 3, 448, 56
H = T // WS
TS = B * C * H
NSEQ = B * H
GDIM = 3 * F
HID = 56
NORM = 1.0 / (NN * F * H * C)


def _cmpx(a, b):
    return jnp.minimum(a, b), jnp.maximum(a, b)


def _median7(v):
    v = list(v)
    for r in range(7):
        pairs = ((0, 1), (2, 3), (4, 5)) if r % 2 == 0 else ((1, 2), (3, 4), (5, 6))
        for i, j in pairs:
            v[i], v[j] = _cmpx(v[i], v[j])
    return v[3]


def _stats_kernel(x_ref, o_ref):
    w = [x_ref[k] for k in range(WS)]
    amax = w[0]
    amin = w[0]
    s = w[0]
    for k in range(1, WS):
        amax = jnp.maximum(amax, w[k])
        amin = jnp.minimum(amin, w[k])
        s = s + w[k]
    mu = s * (1.0 / WS)
    ssd = (w[0] - mu) ** 2
    for k in range(1, WS):
        ssd = ssd + (w[k] - mu) ** 2
    astd = jnp.sqrt(ssd * (1.0 / (WS - 1)))
    sd0 = jnp.sqrt(ssd * (1.0 / WS))
    inv_sd = 1.0 / sd0
    z = [(w[k] - mu) * inv_sd for k in range(WS)]
    zm = z[0]
    for k in range(1, WS):
        zm = zm + z[k]
    zm = zm * (1.0 / WS)
    d = [z[k] - zm for k in range(WS)]
    m2 = d[0] * d[0]
    m3 = d[0] * d[0] * d[0]
    m4 = (d[0] * d[0]) * (d[0] * d[0])
    for k in range(1, WS):
        dk2 = d[k] * d[k]
        m2 = m2 + dk2
        m3 = m3 + dk2 * d[k]
        m4 = m4 + dk2 * dk2
    m2 = m2 * (1.0 / WS)
    m3 = m3 * (1.0 / WS)
    m4 = m4 * (1.0 / WS)
    skew = m3 / jnp.power(m2, 1.5)
    kurt = m4 / (m2 * m2) - 3.0
    med = _median7(w)
    mad = _median7([jnp.abs(w[k] - med) for k in range(WS)])
    o_ref[0] = amax
    o_ref[1] = amin
    o_ref[2] = mu
    o_ref[3] = astd
    o_ref[4] = skew
    o_ref[5] = kurt
    o_ref[6] = mad


def _relieff_kernel(lo_ref, hi_ref, o_ref):
    pb = lo_ref.shape[0]
    i0 = pl.program_id(0)

    xcp = jnp.concatenate([lo_ref[...], hi_ref[...]], axis=2)
    xt = jnp.swapaxes(xcp, 1, 2)

    absdiff = []
    dist = jnp.zeros((pb, F, 2 * F), dtype=jnp.float32)
    for d in range(WS):
        xa = jnp.broadcast_to(xt[:, 0:F, d:d + 1], (pb, F, F))
        xb = jnp.broadcast_to(xt[:, F:2 * F, d:d + 1], (pb, F, F))
        left = jnp.concatenate([xa, xb], axis=2)
        a = jnp.abs(left - xcp[:, d:d + 1, :])
        absdiff.append(a)
        dist = dist + a * a

    big = jnp.float32(3.4e38)
    ii = jax.lax.broadcasted_iota(jnp.int32, (pb, F, 2 * F), 1)
    jj = jax.lax.broadcasted_iota(jnp.int32, (pb, F, 2 * F), 2)
    work = jnp.where(ii == jj % F, big, dist)
    v10 = None
    v20 = None
    for r in range(2 * NN - 1):
        m = jnp.min(work, axis=1, keepdims=True)
        if r == NN - 2:
            v10 = m
        if r == 2 * NN - 2:
            v20 = m
        else:
            work = jnp.where(work <= m, big, work)

    mmask = (dist <= v20).astype(jnp.float32) - 2.0 * (dist <= v10).astype(jnp.float32)

    rows = jax.lax.broadcasted_iota(jnp.int32, (8, 128), 0)
    contrib = jnp.zeros((8, 128), dtype=jnp.float32)
    for d in range(WS):
        sd = jnp.sum(mmask * absdiff[d])
        contrib = jnp.where(rows == d, contrib + sd, contrib)

    @pl.when(i0 == 0)
    def _():
        o_ref[...] = jnp.zeros_like(o_ref)

    o_ref[...] += contrib

    @pl.when(i0 == pl.num_programs(0) - 1)
    def _():
        o_ref[...] = o_ref[...] * NORM


def _gru_kernel(x_ref, wr_ref, wz_ref, wn_ref, br_ref, bz_ref, bn_ref,
                whr_ref, whz_ref, whn_ref, bhr_ref, bhz_ref, bhn_ref,
                wfc_ref, bfc_ref, sc_ref, o_ref, gir, giz, gin):
    x = x_ref[...]
    f32 = jnp.float32

    def tmaj(g):
        return jnp.swapaxes(g.reshape(B, H, HID), 0, 1).reshape(NSEQ, HID)

    gir[...] = tmaj(jnp.dot(x, wr_ref[...], preferred_element_type=f32) + br_ref[...])
    giz[...] = tmaj(jnp.dot(x, wz_ref[...], preferred_element_type=f32) + bz_ref[...])
    gin[...] = tmaj(jnp.dot(x, wn_ref[...], preferred_element_type=f32) + bn_ref[...])

    weff = sc_ref[0, 0] * wfc_ref[0]
    beff = sc_ref[0, 0] * bfc_ref[0:1, :]
    for d in range(1, WS):
        weff = weff + sc_ref[d, 0] * wfc_ref[d]
        beff = beff + sc_ref[d, 0] * bfc_ref[d:d + 1, :]

    whr = whr_ref[...]
    whz = whz_ref[...]
    whn = whn_ref[...]
    bhr = bhr_ref[...]
    bhz = bhz_ref[...]
    bhn = bhn_ref[...]

    def step(t, h):
        gr = jnp.dot(h, whr, preferred_element_type=f32) + bhr
        gz = jnp.dot(h, whz, preferred_element_type=f32) + bhz
        gn = jnp.dot(h, whn, preferred_element_type=f32) + bhn
        r = jax.nn.sigmoid(gir[pl.ds(t * B, B), :] + gr)
        zg = jax.nn.sigmoid(giz[pl.ds(t * B, B), :] + gz)
        n = jnp.tanh(gin[pl.ds(t * B, B), :] + r * gn)
        h = (1.0 - zg) * n + zg * h
        o_ref[pl.ds(t * B, B), :] = (
            jnp.dot(h, weff, preferred_element_type=f32) + beff)
        return h

    h0 = jnp.zeros((B, HID), dtype=f32)
    jax.lax.fori_loop(0, H, step, h0)


@jax.jit
def kernel(x, y, Wc, bc, Wih, Whh, bih, bhh, Wfc, bfc):
    f32 = jnp.float32

    xw = x.reshape(B, C, H, WS, F).transpose(3, 0, 1, 2, 4).reshape(WS, 1344, 128)
    stats = pl.pallas_call(
        _stats_kernel,
        grid=(4,),
        in_specs=[pl.BlockSpec((WS, 336, 128), lambda i: (0, i, 0))],
        out_specs=pl.BlockSpec((WS, 336, 128), lambda i: (0, i, 0)),
        out_shape=jax.ShapeDtypeStruct((WS, 1344, 128), f32),
    )(xw)

    ext_s = stats.reshape(WS, TS, F)
    xcol = ext_s.transpose(1, 0, 2)

    PB = 32
    NBLK = TS // 2 // PB
    scores_pad = pl.pallas_call(
        _relieff_kernel,
        grid=(NBLK,),
        in_specs=[
            pl.BlockSpec((PB, WS, F), lambda i: (i, 0, 0)),
            pl.BlockSpec((PB, WS, F), lambda i: (i + NBLK, 0, 0)),
        ],
        out_specs=pl.BlockSpec((8, 128), lambda i: (0, 0)),
        out_shape=jax.ShapeDtypeStruct((8, 128), f32),
    )(xcol, xcol)

    Wih4 = Wih.reshape(GDIM, C, F, WS)
    Wih_eff = jnp.einsum('gcwd,ck->gkdw', Wih4, Wc).reshape(GDIM, C * F * WS)
    bih_eff = bih + jnp.einsum('gcwd,c->g', Wih4, bc)
    wr = Wih_eff[:HID].T
    wz = Wih_eff[HID:2 * HID].T
    wn = Wih_eff[2 * HID:].T
    br = bih_eff[:HID].reshape(1, HID)
    bz = bih_eff[HID:2 * HID].reshape(1, HID)
    bn = bih_eff[2 * HID:].reshape(1, HID)
    whr = Whh[:HID].T
    whz = Whh[HID:2 * HID].T
    whn = Whh[2 * HID:].T
    bhr = bhh[:HID].reshape(1, HID)
    bhz = bhh[HID:2 * HID].reshape(1, HID)
    bhn = bhh[2 * HID:].reshape(1, HID)
    wfc_t = Wfc.reshape(GDIM, WS, HID).transpose(1, 2, 0)
    bfc_t = bfc.reshape(GDIM, WS).T

    xtm = xcol.reshape(NSEQ, C * F * WS)

    full = lambda s: pl.BlockSpec(s, lambda: tuple(0 for _ in s))
    out_tm = pl.pallas_call(
        _gru_kernel,
        in_specs=[full((NSEQ, C * F * WS)),
                  full((C * F * WS, HID)), full((C * F * WS, HID)), full((C * F * WS, HID)),
                  full((1, HID)), full((1, HID)), full((1, HID)),
                  full((HID, HID)), full((HID, HID)), full((HID, HID)),
                  full((1, HID)), full((1, HID)), full((1, HID)),
                  full((WS, HID, GDIM)), full((WS, GDIM)), full((8, 128))],
        out_specs=full((NSEQ, GDIM)),
        out_shape=jax.ShapeDtypeStruct((NSEQ, GDIM), f32),
        scratch_shapes=[pltpu.VMEM((NSEQ, HID), f32)] * 3,
    )(xtm, wr, wz, wn, br, bz, bn, whr, whz, whn, bhr, bhz, bhn,
      wfc_t, bfc_t, scores_pad)

    return out_tm.reshape(H, B, C, F).transpose(1, 0, 2, 3)
